# Initial kernel scaffold; baseline (speedup 1.0000x reference)
import jax
import jax.numpy as jnp
from jax.experimental import pallas as pl


def kernel(lig_x, lig_h, poc_x, poc_h, lig_edge_index, lig_edge_attr, poc_edge_index, poc_edge_attr, cross_edge_index, cross_edge_attr, params):
    raise NotImplementedError("write your pallas kernel here")


# speedup vs baseline: 2.1859x; 2.1859x over previous
import functools

import jax
import jax.numpy as jnp
from jax import lax
from jax.experimental import pallas as pl
from jax.experimental.pallas import tpu as pltpu
from jax.experimental.pallas import tpu_sc as plsc

N = 10000
NT = 2 * N
E = 160000
E3 = 3 * E
DH = 128
XP = 16
L = 2

NC, NS = 2, 16
NW = NC * NS

G_PW = E3 // NW
G_CH = 600
S_CE = 200
S_PW0 = (2 * E) // NS
S_PW1 = E // NS
STRIPE = N // NS

BLK = 800
BN = 1000

def _sc_mesh():
    return plsc.VectorSubcoreMesh(core_axis_name="c", subcore_axis_name="s",
                                  num_cores=NC, num_subcores=NS)


def _silu(v):
    return v * jax.nn.sigmoid(v)


def _sc_gather(H, X, IDXD, IDXS):
    @functools.partial(
        pl.kernel,
        out_type=[
            jax.ShapeDtypeStruct((E3, DH), jnp.float32),
            jax.ShapeDtypeStruct((E3, DH), jnp.float32),
            jax.ShapeDtypeStruct((E3, XP), jnp.float32),
            jax.ShapeDtypeStruct((E3, XP), jnp.float32),
        ],
        mesh=_sc_mesh(),
        compiler_params=pltpu.CompilerParams(use_tc_tiling_on_sc=False),
        scratch_types=[
            pltpu.VMEM((G_PW,), jnp.int32),
            pltpu.VMEM((G_CH, DH), jnp.float32),
            pltpu.VMEM((G_CH, XP), jnp.float32),
        ],
    )
    def k(h_hbm, x_hbm, idxd_hbm, idxs_hbm, hd_out, hs_out, xd_out, xs_out,
          idx_v, hbuf, xbuf):
        wid = lax.axis_index("s") * NC + lax.axis_index("c")
        base = wid * G_PW
        for idxh, tasks in (
            (idxd_hbm, ((h_hbm, hd_out, hbuf), (x_hbm, xd_out, xbuf))),
            (idxs_hbm, ((h_hbm, hs_out, hbuf), (x_hbm, xs_out, xbuf))),
        ):
            pltpu.sync_copy(idxh.at[pl.ds(base, G_PW)], idx_v)
            for src, out, buf in tasks:
                def body(j, carry, src=src, out=out, buf=buf):
                    off = j * G_CH
                    pltpu.sync_copy(src.at[idx_v.at[pl.ds(off, G_CH)]], buf)
                    pltpu.sync_copy(buf, out.at[pl.ds(base + off, G_CH)])
                    return carry
                lax.fori_loop(0, G_PW // G_CH, body, 0)

    return k(H, X, IDXD, IDXS)


def _sc_scatter(M, RC, SIDX_LC, SIDX_P, Z, Zx):
    @functools.partial(
        pl.kernel,
        out_type=[
            jax.ShapeDtypeStruct((NT, DH), jnp.float32),
            jax.ShapeDtypeStruct((NT, XP), jnp.float32),
        ],
        mesh=_sc_mesh(),
        compiler_params=pltpu.CompilerParams(use_tc_tiling_on_sc=False),
        scratch_types=[
            pltpu.VMEM_SHARED((N, DH), jnp.float32),
            pltpu.VMEM_SHARED((N, XP), jnp.float32),
            pltpu.VMEM((S_CE, DH), jnp.float32),
            pltpu.VMEM((S_CE, XP), jnp.float32),
            pltpu.VMEM((S_CE,), jnp.int32),
        ],
    )
    def k(m_hbm, rc_hbm, silc_hbm, sip_hbm, z_hbm, zx_hbm,
          agg_out, xupd_out, aggS, xupdS, mbuf, rcbuf, idxbuf):
        cid = lax.axis_index("c")
        sid = lax.axis_index("s")
        row0 = sid * STRIPE
        pltpu.sync_copy(z_hbm.at[pl.ds(row0, STRIPE)], aggS.at[pl.ds(row0, STRIPE)])
        pltpu.sync_copy(zx_hbm.at[pl.ds(row0, STRIPE)], xupdS.at[pl.ds(row0, STRIPE)])
        plsc.subcore_barrier()

        @pl.when(cid == 0)
        def _():
            base = sid * S_PW0
            def body(j, carry):
                off = base + j * S_CE
                pltpu.sync_copy(silc_hbm.at[pl.ds(off, S_CE)], idxbuf)
                pltpu.sync_copy(m_hbm.at[pl.ds(off, S_CE)], mbuf)
                pltpu.sync_copy(mbuf, aggS.at[idxbuf], add=True)
                pltpu.sync_copy(rc_hbm.at[pl.ds(off, S_CE)], rcbuf)
                pltpu.sync_copy(rcbuf, xupdS.at[idxbuf], add=True)
                return carry
            lax.fori_loop(0, S_PW0 // S_CE, body, 0)

        @pl.when(cid == 1)
        def _():
            base = sid * S_PW1
            def body(j, carry):
                off = base + j * S_CE
                pltpu.sync_copy(sip_hbm.at[pl.ds(off, S_CE)], idxbuf)
                pltpu.sync_copy(m_hbm.at[pl.ds(2 * E + off, S_CE)], mbuf)
                pltpu.sync_copy(mbuf, aggS.at[idxbuf], add=True)
                return carry
            lax.fori_loop(0, S_PW1 // S_CE, body, 0)

        plsc.subcore_barrier()
        pltpu.sync_copy(aggS.at[pl.ds(row0, STRIPE)],
                        agg_out.at[pl.ds(cid * N + row0, STRIPE)])
        pltpu.sync_copy(xupdS.at[pl.ds(row0, STRIPE)],
                        xupd_out.at[pl.ds(cid * N + row0, STRIPE)])

    return k(M, RC, SIDX_LC, SIDX_P, Z, Zx)


def _tc_prologue(H0, Win, BIN, TADD):
    def body(h0_ref, win_ref, bin_ref, tadd_ref, h_ref):
        u = jnp.dot(h0_ref[...], win_ref[0], preferred_element_type=jnp.float32)
        h_ref[...] = _silu(u + bin_ref[0]) + tadd_ref[0]

    nb = N // BN
    return pl.pallas_call(
        body,
        grid=(NT // BN,),
        in_specs=[
            pl.BlockSpec((BN, DH), lambda i: (i, 0)),
            pl.BlockSpec((1, DH, DH), lambda i: (i // nb, 0, 0)),
            pl.BlockSpec((1, 1, DH), lambda i: (i // nb, 0, 0)),
            pl.BlockSpec((1, 1, DH), lambda i: (i // nb, 0, 0)),
        ],
        out_specs=pl.BlockSpec((BN, DH), lambda i: (i, 0)),
        out_shape=jax.ShapeDtypeStruct((NT, DH), jnp.float32),
    )(H0, Win, BIN, TADD)


def _tc_edge(HD, HS, XD, XS, EA, Wd, Ws, Wdist, Wea, Be, W2, B2, Wx):
    def body(hd_ref, hs_ref, xd_ref, xs_ref, ea_ref,
             wd_ref, ws_ref, wdist_ref, wea_ref, be_ref, w2_ref, b2_ref,
             wx_ref, m_ref, rc_ref):
        rel = xd_ref[...] - xs_ref[...]
        d2 = jnp.sum(rel * rel, axis=1, keepdims=True)
        m1 = (jnp.dot(hd_ref[...], wd_ref[0], preferred_element_type=jnp.float32)
              + jnp.dot(hs_ref[...], ws_ref[0], preferred_element_type=jnp.float32)
              + jnp.dot(ea_ref[...], wea_ref[0], preferred_element_type=jnp.float32)
              + d2 * wdist_ref[0] + be_ref[0])
        m1 = _silu(m1)
        m2 = _silu(jnp.dot(m1, w2_ref[0], preferred_element_type=jnp.float32)
                   + b2_ref[0])
        coef = jnp.sum(m2 * wx_ref[0], axis=1, keepdims=True)
        m_ref[...] = m2
        rc_ref[...] = rel * coef

    eb = E // BLK
    em = lambda t, b: (t * eb + b, 0)
    wm3 = lambda t, b: (t, 0, 0)
    return pl.pallas_call(
        body,
        grid=(3, eb),
        in_specs=[
            pl.BlockSpec((BLK, DH), em),
            pl.BlockSpec((BLK, DH), em),
            pl.BlockSpec((BLK, XP), em),
            pl.BlockSpec((BLK, XP), em),
            pl.BlockSpec((BLK, XP), em),
            pl.BlockSpec((1, DH, DH), wm3),
            pl.BlockSpec((1, DH, DH), wm3),
            pl.BlockSpec((1, 1, DH), wm3),
            pl.BlockSpec((1, XP, DH), wm3),
            pl.BlockSpec((1, 1, DH), wm3),
            pl.BlockSpec((1, DH, DH), wm3),
            pl.BlockSpec((1, 1, DH), wm3),
            pl.BlockSpec((1, 1, DH), wm3),
        ],
        out_specs=[
            pl.BlockSpec((BLK, DH), em),
            pl.BlockSpec((BLK, XP), em),
        ],
        out_shape=[
            jax.ShapeDtypeStruct((E3, DH), jnp.float32),
            jax.ShapeDtypeStruct((E3, XP), jnp.float32),
        ],
    )(HD, HS, XD, XS, EA, Wd, Ws, Wdist, Wea, Be, W2, B2, Wx)


def _tc_node(H, AGG, X, XUPD, Wn1, Wn2, Bn):
    def body(h_ref, agg_ref, x_ref, xu_ref, wn1_ref, wn2_ref, bn_ref,
             hn_ref, xn_ref):
        h = h_ref[...]
        u = (jnp.dot(h, wn1_ref[0], preferred_element_type=jnp.float32)
             + jnp.dot(agg_ref[...], wn2_ref[0], preferred_element_type=jnp.float32)
             + bn_ref[0])
        hn_ref[...] = h + _silu(u)
        xn_ref[...] = x_ref[...] + xu_ref[...] * (1.0 / 32.0)

    nb = N // BN
    return pl.pallas_call(
        body,
        grid=(NT // BN,),
        in_specs=[
            pl.BlockSpec((BN, DH), lambda i: (i, 0)),
            pl.BlockSpec((BN, DH), lambda i: (i, 0)),
            pl.BlockSpec((BN, XP), lambda i: (i, 0)),
            pl.BlockSpec((BN, XP), lambda i: (i, 0)),
            pl.BlockSpec((1, DH, DH), lambda i: (i // nb, 0, 0)),
            pl.BlockSpec((1, DH, DH), lambda i: (i // nb, 0, 0)),
            pl.BlockSpec((1, 1, DH), lambda i: (i // nb, 0, 0)),
        ],
        out_specs=[
            pl.BlockSpec((BN, DH), lambda i: (i, 0)),
            pl.BlockSpec((BN, XP), lambda i: (i, 0)),
        ],
        out_shape=[
            jax.ShapeDtypeStruct((NT, DH), jnp.float32),
            jax.ShapeDtypeStruct((NT, XP), jnp.float32),
        ],
    )(H, AGG, X, XUPD, Wn1, Wn2, Bn)


def _tc_final(H, X, C, Wout, Bout):
    def body(h_ref, x_ref, c_ref, wout_ref, bout_ref, out_ref):
        @pl.when(pl.program_id(0) == 0)
        def _():
            out_ref[...] = jnp.zeros_like(out_ref)
        d = (x_ref[...] - c_ref[...]
             + jnp.dot(h_ref[...], wout_ref[...], preferred_element_type=jnp.float32)
             + bout_ref[...])
        out_ref[...] = out_ref[...] + jnp.sum(d * d) * (1.0 / (3 * N))

    return pl.pallas_call(
        body,
        grid=(N // BN,),
        in_specs=[
            pl.BlockSpec((BN, DH), lambda i: (i, 0)),
            pl.BlockSpec((BN, XP), lambda i: (i, 0)),
            pl.BlockSpec((BN, XP), lambda i: (i, 0)),
            pl.BlockSpec((DH, XP), lambda i: (0, 0)),
            pl.BlockSpec((1, XP), lambda i: (0, 0)),
        ],
        out_specs=pl.BlockSpec((1, 1), lambda i: (0, 0)),
        out_shape=jax.ShapeDtypeStruct((1, 1), jnp.float32),
    )(H, X, C, Wout, Bout)


def kernel(lig_x, lig_h, poc_x, poc_h, lig_edge_index, lig_edge_attr,
           poc_edge_index, poc_edge_attr, cross_edge_index, cross_edge_attr,
           params):
    p = params
    k1, k2 = jax.random.split(jax.random.key(42))
    t = jax.random.uniform(k1, (), dtype=jnp.float32)
    x0 = jax.random.normal(k2, (N, 3), dtype=jnp.float32)
    poc_center = jnp.mean(poc_x, axis=0, keepdims=True)
    lig_x1 = lig_x - poc_center
    poc_xc = poc_x - poc_center
    x_t = (1.0 - t) * x0 + t * lig_x1

    lig_src, lig_dst = lig_edge_index[0], lig_edge_index[1]
    poc_src, poc_dst = poc_edge_index[0], poc_edge_index[1]
    poc_idx, lig_idx = cross_edge_index[0], cross_edge_index[1]
    IDXD = jnp.concatenate([lig_dst, lig_idx, poc_dst + N]).astype(jnp.int32)
    IDXS = jnp.concatenate([lig_src, poc_idx + N, poc_src + N]).astype(jnp.int32)
    SIDX_LC = jnp.concatenate([lig_dst, lig_idx]).astype(jnp.int32)
    SIDX_P = poc_dst.astype(jnp.int32)
    EA = jnp.concatenate([lig_edge_attr, cross_edge_attr, poc_edge_attr], axis=0)

    We = jnp.stack([p['We_lig'], p['We_cross'], p['We_poc']], axis=1)
    WD, WS = We[:, :, 0:DH, :], We[:, :, DH:2 * DH, :]
    WDIST = We[:, :, 2 * DH:2 * DH + 1, :]
    WEA = We[:, :, 2 * DH + 1:, :]
    BE = jnp.stack([p['be_lig'], p['be_cross'], p['be_poc']], axis=1)[:, :, None, :]
    W2 = jnp.stack([p['We2_lig'], p['We2_cross'], p['We2_poc']], axis=1)
    B2 = jnp.stack([p['be2_lig'], p['be2_cross'], p['be2_poc']], axis=1)[:, :, None, :]
    wx_l = jnp.swapaxes(p['Wx_lig'], 1, 2)
    wx_c = jnp.swapaxes(p['Wx_cross'], 1, 2)
    WX = jnp.stack([wx_l, wx_c, jnp.zeros_like(wx_l)], axis=1)
    Wn = jnp.stack([p['Wn_lig'], p['Wn_poc']], axis=1)
    WN1, WN2 = Wn[:, :, 0:DH, :], Wn[:, :, DH:, :]
    BN_ = jnp.stack([p['bn_lig'], p['bn_poc']], axis=1)[:, :, None, :]

    Win = jnp.stack([p['W_lig_in'], p['W_poc_in']])
    BIN = jnp.stack([p['b_lig_in'], p['b_poc_in']])[:, None, :]
    t_emb = _silu(t * p['W_t'] + p['b_t'])
    TADD = jnp.stack([t_emb, jnp.zeros_like(t_emb)])

    H0 = jnp.concatenate([lig_h, poc_h], axis=0)
    X = jnp.concatenate([
        jnp.pad(x_t, ((0, 0), (0, XP - 3))),
        jnp.pad(poc_xc, ((0, 0), (0, XP - 3))),
    ], axis=0)
    Z = jnp.zeros((N, DH), jnp.float32)
    Zx = jnp.zeros((N, XP), jnp.float32)

    H = _tc_prologue(H0, Win, BIN, TADD)
    for l in range(L):
        HD, HS, XD, XS = _sc_gather(H, X, IDXD, IDXS)
        M, RC = _tc_edge(HD, HS, XD, XS, EA, WD[l], WS[l], WDIST[l], WEA[l],
                         BE[l], W2[l], B2[l], WX[l])
        AGG, XUPD = _sc_scatter(M, RC, SIDX_LC, SIDX_P, Z, Zx)
        H, X = _tc_node(H, AGG, X, XUPD, WN1[l], WN2[l], BN_[l])

    C = jnp.pad(x_t + (lig_x1 - x0), ((0, 0), (0, XP - 3)))
    Wout = jnp.pad(p['W_out'], ((0, 0), (0, XP - 3)))
    Bout = jnp.pad(p['b_out'], (0, XP - 3))[None, :]
    loss = _tc_final(H, X, C, Wout, Bout)
    return loss[0, 0]


# speedup vs baseline: 2.3658x; 1.0823x over previous
import functools

import jax
import jax.numpy as jnp
from jax import lax
from jax.experimental import pallas as pl
from jax.experimental.pallas import tpu as pltpu
from jax.experimental.pallas import tpu_sc as plsc

N = 10000
NT = 2 * N
E = 160000
E3 = 3 * E
DH = 128
XP = 16
L = 2

NC, NS = 2, 16
NW = NC * NS

G_PW = E3 // NW
G_CH = 200
G_NCH = G_PW // G_CH
S_CE = 80
S_PW0 = (2 * E) // NS
S_PW1 = E // NS
STRIPE = N // NS

BLK = 800
BN = 1000

def _sc_mesh():
    return plsc.VectorSubcoreMesh(core_axis_name="c", subcore_axis_name="s",
                                  num_cores=NC, num_subcores=NS)


def _silu(v):
    return v * jax.nn.sigmoid(v)


def _sc_gather(H, X, IDXD, IDXS):
    @functools.partial(
        pl.kernel,
        out_type=[
            jax.ShapeDtypeStruct((E3, DH), jnp.float32),
            jax.ShapeDtypeStruct((E3, DH), jnp.float32),
            jax.ShapeDtypeStruct((E3, XP), jnp.float32),
            jax.ShapeDtypeStruct((E3, XP), jnp.float32),
        ],
        mesh=_sc_mesh(),
        compiler_params=pltpu.CompilerParams(use_tc_tiling_on_sc=False),
        scratch_types=[
            pltpu.VMEM((G_PW,), jnp.int32),
            pltpu.VMEM((G_CH, DH), jnp.float32),
            pltpu.VMEM((G_CH, DH), jnp.float32),
            pltpu.VMEM((G_CH, XP), jnp.float32),
            pltpu.VMEM((G_CH, XP), jnp.float32),
        ] + [pltpu.SemaphoreType.DMA] * 8,
    )
    def k(h_hbm, x_hbm, idxd_hbm, idxs_hbm, hd_out, hs_out, xd_out, xs_out,
          idx_v, hb0, hb1, xb0, xb1,
          gh0, gh1, gx0, gx1, sh0, sh1, sx0, sx1):
        wid = lax.axis_index("s") * NC + lax.axis_index("c")
        base = wid * G_PW
        for idxh, hout, xout in ((idxd_hbm, hd_out, xd_out),
                                 (idxs_hbm, hs_out, xs_out)):
            pltpu.sync_copy(idxh.at[pl.ds(base, G_PW)], idx_v)

            def fire_g(j, hb, xb, ghs, gxs):
                ix = idx_v.at[pl.ds(j * G_CH, G_CH)]
                pltpu.async_copy(h_hbm.at[ix], hb, ghs)
                pltpu.async_copy(x_hbm.at[ix], xb, gxs)

            def fire_s(j, hb, xb, shs, sxs, hout=hout, xout=xout):
                pltpu.async_copy(hb, hout.at[pl.ds(base + j * G_CH, G_CH)], shs)
                pltpu.async_copy(xb, xout.at[pl.ds(base + j * G_CH, G_CH)], sxs)

            ix0 = idx_v.at[pl.ds(0, G_CH)]
            o0 = pl.ds(base, G_CH)
            wgh0 = pltpu.make_async_copy(h_hbm.at[ix0], hb0, gh0)
            wgh1 = pltpu.make_async_copy(h_hbm.at[ix0], hb1, gh1)
            wgx0 = pltpu.make_async_copy(x_hbm.at[ix0], xb0, gx0)
            wgx1 = pltpu.make_async_copy(x_hbm.at[ix0], xb1, gx1)
            wsh0 = pltpu.make_async_copy(hb0, hout.at[o0], sh0)
            wsh1 = pltpu.make_async_copy(hb1, hout.at[o0], sh1)
            wsx0 = pltpu.make_async_copy(xb0, xout.at[o0], sx0)
            wsx1 = pltpu.make_async_copy(xb1, xout.at[o0], sx1)

            fire_g(0, hb0, xb0, gh0, gx0)
            fire_g(1, hb1, xb1, gh1, gx1)

            def body(jj, carry):
                j0 = 2 * jj
                j1 = j0 + 1
                wgh0.wait(); wgx0.wait()
                fire_s(j0, hb0, xb0, sh0, sx0)
                wgh1.wait(); wgx1.wait()
                fire_s(j1, hb1, xb1, sh1, sx1)
                wsh0.wait(); wsx0.wait()

                @pl.when(j0 + 2 < G_NCH)
                def _():
                    fire_g(j0 + 2, hb0, xb0, gh0, gx0)
                wsh1.wait(); wsx1.wait()

                @pl.when(j1 + 2 < G_NCH)
                def _():
                    fire_g(j1 + 2, hb1, xb1, gh1, gx1)
                return carry

            lax.fori_loop(0, G_NCH // 2, body, 0)
            wgh0.wait(); wgx0.wait()
            fire_s(G_NCH - 1, hb0, xb0, sh0, sx0)
            wsh0.wait(); wsx0.wait()

    return k(H, X, IDXD, IDXS)


def _sc_scatter(M, RC, SIDX_LC, SIDX_P, Z, Zx):
    @functools.partial(
        pl.kernel,
        out_type=[
            jax.ShapeDtypeStruct((NT, DH), jnp.float32),
            jax.ShapeDtypeStruct((NT, XP), jnp.float32),
        ],
        mesh=_sc_mesh(),
        compiler_params=pltpu.CompilerParams(use_tc_tiling_on_sc=False),
        scratch_types=[
            pltpu.VMEM_SHARED((N, DH), jnp.float32),
            pltpu.VMEM_SHARED((N, XP), jnp.float32),
            pltpu.VMEM((S_CE, DH), jnp.float32),
            pltpu.VMEM((S_CE, DH), jnp.float32),
            pltpu.VMEM((S_CE, XP), jnp.float32),
            pltpu.VMEM((S_CE, XP), jnp.float32),
            pltpu.VMEM((S_CE,), jnp.int32),
            pltpu.VMEM((S_CE,), jnp.int32),
        ] + [pltpu.SemaphoreType.DMA] * 4,
    )
    def k(m_hbm, rc_hbm, silc_hbm, sip_hbm, z_hbm, zx_hbm,
          agg_out, xupd_out, aggS, xupdS, mb0, mb1, rb0, rb1, ib0, ib1,
          ls0, ls1, ss0, ss1):
        cid = lax.axis_index("c")
        sid = lax.axis_index("s")
        row0 = sid * STRIPE
        pltpu.sync_copy(z_hbm.at[pl.ds(row0, STRIPE)], aggS.at[pl.ds(row0, STRIPE)])
        pltpu.sync_copy(zx_hbm.at[pl.ds(row0, STRIPE)], xupdS.at[pl.ds(row0, STRIPE)])
        plsc.subcore_barrier()

        def stream(src_hbm, idx_hbm, tgt, b0, b1, src_base, idx_base, nch):
            def fire_l(j, b, ib, ls):
                pltpu.async_copy(idx_hbm.at[pl.ds(idx_base + j * S_CE, S_CE)], ib, ls)
                pltpu.async_copy(src_hbm.at[pl.ds(src_base + j * S_CE, S_CE)], b, ls)

            i0 = pl.ds(idx_base, S_CE)
            s0 = pl.ds(src_base, S_CE)
            wl0i = pltpu.make_async_copy(idx_hbm.at[i0], ib0, ls0)
            wl0m = pltpu.make_async_copy(src_hbm.at[s0], b0, ls0)
            wl1i = pltpu.make_async_copy(idx_hbm.at[i0], ib1, ls1)
            wl1m = pltpu.make_async_copy(src_hbm.at[s0], b1, ls1)
            wsc0 = pltpu.make_async_copy(b0, tgt.at[ib0], ss0)
            wsc1 = pltpu.make_async_copy(b1, tgt.at[ib1], ss1)

            fire_l(0, b0, ib0, ls0)
            fire_l(1, b1, ib1, ls1)

            def body(jj, carry):
                j0 = 2 * jj
                wl0i.wait(); wl0m.wait()
                pltpu.async_copy(b0, tgt.at[ib0], ss0, add=True)
                wl1i.wait(); wl1m.wait()
                pltpu.async_copy(b1, tgt.at[ib1], ss1, add=True)
                wsc0.wait()

                @pl.when(j0 + 2 < nch)
                def _():
                    fire_l(j0 + 2, b0, ib0, ls0)
                wsc1.wait()

                @pl.when(j0 + 3 < nch)
                def _():
                    fire_l(j0 + 3, b1, ib1, ls1)
                return carry

            lax.fori_loop(0, nch // 2, body, 0)
            if nch % 2:
                wl0i.wait(); wl0m.wait()
                pltpu.async_copy(b0, tgt.at[ib0], ss0, add=True)
                wsc0.wait()

        @pl.when(cid == 0)
        def _():
            stream(m_hbm, silc_hbm, aggS, mb0, mb1,
                   sid * S_PW0, sid * S_PW0, S_PW0 // S_CE)

        @pl.when(cid == 1)
        def _():
            stream(m_hbm, sip_hbm, aggS, mb0, mb1,
                   2 * E + sid * S_PW1, sid * S_PW1, S_PW1 // S_CE)
            stream(rc_hbm, silc_hbm, xupdS, rb0, rb1,
                   sid * S_PW0, sid * S_PW0, S_PW0 // S_CE)

        plsc.subcore_barrier()
        pltpu.sync_copy(aggS.at[pl.ds(row0, STRIPE)],
                        agg_out.at[pl.ds(cid * N + row0, STRIPE)])
        pltpu.sync_copy(xupdS.at[pl.ds(row0, STRIPE)],
                        xupd_out.at[pl.ds((1 - cid) * N + row0, STRIPE)])

    return k(M, RC, SIDX_LC, SIDX_P, Z, Zx)


def _tc_prologue(H0, Win, BIN, TADD):
    def body(h0_ref, win_ref, bin_ref, tadd_ref, h_ref):
        u = jnp.dot(h0_ref[...], win_ref[0], preferred_element_type=jnp.float32)
        h_ref[...] = _silu(u + bin_ref[0]) + tadd_ref[0]

    nb = N // BN
    return pl.pallas_call(
        body,
        grid=(NT // BN,),
        in_specs=[
            pl.BlockSpec((BN, DH), lambda i: (i, 0)),
            pl.BlockSpec((1, DH, DH), lambda i: (i // nb, 0, 0)),
            pl.BlockSpec((1, 1, DH), lambda i: (i // nb, 0, 0)),
            pl.BlockSpec((1, 1, DH), lambda i: (i // nb, 0, 0)),
        ],
        out_specs=pl.BlockSpec((BN, DH), lambda i: (i, 0)),
        out_shape=jax.ShapeDtypeStruct((NT, DH), jnp.float32),
    )(H0, Win, BIN, TADD)


def _tc_edge(HD, HS, XD, XS, EA, Wd, Ws, Wdist, Wea, Be, W2, B2, Wx):
    def body(hd_ref, hs_ref, xd_ref, xs_ref, ea_ref,
             wd_ref, ws_ref, wdist_ref, wea_ref, be_ref, w2_ref, b2_ref,
             wx_ref, m_ref, rc_ref):
        rel = xd_ref[...] - xs_ref[...]
        d2 = jnp.sum(rel * rel, axis=1, keepdims=True)
        m1 = (jnp.dot(hd_ref[...], wd_ref[0], preferred_element_type=jnp.float32)
              + jnp.dot(hs_ref[...], ws_ref[0], preferred_element_type=jnp.float32)
              + jnp.dot(ea_ref[...], wea_ref[0], preferred_element_type=jnp.float32)
              + d2 * wdist_ref[0] + be_ref[0])
        m1 = _silu(m1)
        m2 = _silu(jnp.dot(m1, w2_ref[0], preferred_element_type=jnp.float32)
                   + b2_ref[0])
        coef = jnp.sum(m2 * wx_ref[0], axis=1, keepdims=True)
        m_ref[...] = m2
        rc_ref[...] = rel * coef

    eb = E // BLK
    em = lambda t, b: (t * eb + b, 0)
    wm3 = lambda t, b: (t, 0, 0)
    return pl.pallas_call(
        body,
        grid=(3, eb),
        in_specs=[
            pl.BlockSpec((BLK, DH), em),
            pl.BlockSpec((BLK, DH), em),
            pl.BlockSpec((BLK, XP), em),
            pl.BlockSpec((BLK, XP), em),
            pl.BlockSpec((BLK, XP), em),
            pl.BlockSpec((1, DH, DH), wm3),
            pl.BlockSpec((1, DH, DH), wm3),
            pl.BlockSpec((1, 1, DH), wm3),
            pl.BlockSpec((1, XP, DH), wm3),
            pl.BlockSpec((1, 1, DH), wm3),
            pl.BlockSpec((1, DH, DH), wm3),
            pl.BlockSpec((1, 1, DH), wm3),
            pl.BlockSpec((1, 1, DH), wm3),
        ],
        out_specs=[
            pl.BlockSpec((BLK, DH), em),
            pl.BlockSpec((BLK, XP), em),
        ],
        out_shape=[
            jax.ShapeDtypeStruct((E3, DH), jnp.float32),
            jax.ShapeDtypeStruct((E3, XP), jnp.float32),
        ],
    )(HD, HS, XD, XS, EA, Wd, Ws, Wdist, Wea, Be, W2, B2, Wx)


def _tc_node(H, AGG, X, XUPD, Wn1, Wn2, Bn):
    def body(h_ref, agg_ref, x_ref, xu_ref, wn1_ref, wn2_ref, bn_ref,
             hn_ref, xn_ref):
        h = h_ref[...]
        u = (jnp.dot(h, wn1_ref[0], preferred_element_type=jnp.float32)
             + jnp.dot(agg_ref[...], wn2_ref[0], preferred_element_type=jnp.float32)
             + bn_ref[0])
        hn_ref[...] = h + _silu(u)
        xn_ref[...] = x_ref[...] + xu_ref[...] * (1.0 / 32.0)

    nb = N // BN
    return pl.pallas_call(
        body,
        grid=(NT // BN,),
        in_specs=[
            pl.BlockSpec((BN, DH), lambda i: (i, 0)),
            pl.BlockSpec((BN, DH), lambda i: (i, 0)),
            pl.BlockSpec((BN, XP), lambda i: (i, 0)),
            pl.BlockSpec((BN, XP), lambda i: (i, 0)),
            pl.BlockSpec((1, DH, DH), lambda i: (i // nb, 0, 0)),
            pl.BlockSpec((1, DH, DH), lambda i: (i // nb, 0, 0)),
            pl.BlockSpec((1, 1, DH), lambda i: (i // nb, 0, 0)),
        ],
        out_specs=[
            pl.BlockSpec((BN, DH), lambda i: (i, 0)),
            pl.BlockSpec((BN, XP), lambda i: (i, 0)),
        ],
        out_shape=[
            jax.ShapeDtypeStruct((NT, DH), jnp.float32),
            jax.ShapeDtypeStruct((NT, XP), jnp.float32),
        ],
    )(H, AGG, X, XUPD, Wn1, Wn2, Bn)


def _tc_final(H, X, C, Wout, Bout):
    def body(h_ref, x_ref, c_ref, wout_ref, bout_ref, out_ref):
        @pl.when(pl.program_id(0) == 0)
        def _():
            out_ref[...] = jnp.zeros_like(out_ref)
        d = (x_ref[...] - c_ref[...]
             + jnp.dot(h_ref[...], wout_ref[...], preferred_element_type=jnp.float32)
             + bout_ref[...])
        out_ref[...] = out_ref[...] + jnp.sum(d * d) * (1.0 / (3 * N))

    return pl.pallas_call(
        body,
        grid=(N // BN,),
        in_specs=[
            pl.BlockSpec((BN, DH), lambda i: (i, 0)),
            pl.BlockSpec((BN, XP), lambda i: (i, 0)),
            pl.BlockSpec((BN, XP), lambda i: (i, 0)),
            pl.BlockSpec((DH, XP), lambda i: (0, 0)),
            pl.BlockSpec((1, XP), lambda i: (0, 0)),
        ],
        out_specs=pl.BlockSpec((1, 1), lambda i: (0, 0)),
        out_shape=jax.ShapeDtypeStruct((1, 1), jnp.float32),
    )(H, X, C, Wout, Bout)


def kernel(lig_x, lig_h, poc_x, poc_h, lig_edge_index, lig_edge_attr,
           poc_edge_index, poc_edge_attr, cross_edge_index, cross_edge_attr,
           params):
    p = params
    k1, k2 = jax.random.split(jax.random.key(42))
    t = jax.random.uniform(k1, (), dtype=jnp.float32)
    x0 = jax.random.normal(k2, (N, 3), dtype=jnp.float32)
    poc_center = jnp.mean(poc_x, axis=0, keepdims=True)
    lig_x1 = lig_x - poc_center
    poc_xc = poc_x - poc_center
    x_t = (1.0 - t) * x0 + t * lig_x1

    lig_src, lig_dst = lig_edge_index[0], lig_edge_index[1]
    poc_src, poc_dst = poc_edge_index[0], poc_edge_index[1]
    poc_idx, lig_idx = cross_edge_index[0], cross_edge_index[1]
    IDXD = jnp.concatenate([lig_dst, lig_idx, poc_dst + N]).astype(jnp.int32)
    IDXS = jnp.concatenate([lig_src, poc_idx + N, poc_src + N]).astype(jnp.int32)
    SIDX_LC = jnp.concatenate([lig_dst, lig_idx]).astype(jnp.int32)
    SIDX_P = poc_dst.astype(jnp.int32)
    EA = jnp.concatenate([lig_edge_attr, cross_edge_attr, poc_edge_attr], axis=0)

    We = jnp.stack([p['We_lig'], p['We_cross'], p['We_poc']], axis=1)
    WD, WS = We[:, :, 0:DH, :], We[:, :, DH:2 * DH, :]
    WDIST = We[:, :, 2 * DH:2 * DH + 1, :]
    WEA = We[:, :, 2 * DH + 1:, :]
    BE = jnp.stack([p['be_lig'], p['be_cross'], p['be_poc']], axis=1)[:, :, None, :]
    W2 = jnp.stack([p['We2_lig'], p['We2_cross'], p['We2_poc']], axis=1)
    B2 = jnp.stack([p['be2_lig'], p['be2_cross'], p['be2_poc']], axis=1)[:, :, None, :]
    wx_l = jnp.swapaxes(p['Wx_lig'], 1, 2)
    wx_c = jnp.swapaxes(p['Wx_cross'], 1, 2)
    WX = jnp.stack([wx_l, wx_c, jnp.zeros_like(wx_l)], axis=1)
    Wn = jnp.stack([p['Wn_lig'], p['Wn_poc']], axis=1)
    WN1, WN2 = Wn[:, :, 0:DH, :], Wn[:, :, DH:, :]
    BN_ = jnp.stack([p['bn_lig'], p['bn_poc']], axis=1)[:, :, None, :]

    Win = jnp.stack([p['W_lig_in'], p['W_poc_in']])
    BIN = jnp.stack([p['b_lig_in'], p['b_poc_in']])[:, None, :]
    t_emb = _silu(t * p['W_t'] + p['b_t'])
    TADD = jnp.stack([t_emb, jnp.zeros_like(t_emb)])

    H0 = jnp.concatenate([lig_h, poc_h], axis=0)
    X = jnp.concatenate([
        jnp.pad(x_t, ((0, 0), (0, XP - 3))),
        jnp.pad(poc_xc, ((0, 0), (0, XP - 3))),
    ], axis=0)
    Z = jnp.zeros((N, DH), jnp.float32)
    Zx = jnp.zeros((N, XP), jnp.float32)

    H = _tc_prologue(H0, Win, BIN, TADD)
    for l in range(L):
        HD, HS, XD, XS = _sc_gather(H, X, IDXD, IDXS)
        M, RC = _tc_edge(HD, HS, XD, XS, EA, WD[l], WS[l], WDIST[l], WEA[l],
                         BE[l], W2[l], B2[l], WX[l])
        AGG, XUPD = _sc_scatter(M, RC, SIDX_LC, SIDX_P, Z, Zx)
        H, X = _tc_node(H, AGG, X, XUPD, WN1[l], WN2[l], BN_[l])

    C = jnp.pad(x_t + (lig_x1 - x0), ((0, 0), (0, XP - 3)))
    Wout = jnp.pad(p['W_out'], ((0, 0), (0, XP - 3)))
    Bout = jnp.pad(p['b_out'], (0, XP - 3))[None, :]
    loss = _tc_final(H, X, C, Wout, Bout)
    return loss[0, 0]
